# baseline (speedup 1.0000x reference)
import functools

import jax
import jax.numpy as jnp
from jax import lax
from jax.experimental import pallas as pl
from jax.experimental.pallas import tpu as pltpu
from jax.experimental.pallas import tpu_sc as plsc

ROWS = 4096
COLS = 200
EMBED_DIM = 32
B = ROWS * COLS

_NUM_CORES = 2
_NUM_SUBCORES = 16
NW = _NUM_CORES * _NUM_SUBCORES

UNIT = 512
IHR = UNIT // 128
NDH = EMBED_DIM // 8
UNIT_OUT = UNIT * EMBED_DIM
SEG = UNIT_OUT // NDH
UNITS_PER_COL = ROWS // UNIT
NUNITS = COLS * UNITS_PER_COL
UNITS_PER_W = NUNITS // NW


def _make_gather():
    mesh = plsc.VectorSubcoreMesh(core_axis_name="c", subcore_axis_name="s")

    @functools.partial(
        pl.kernel,
        mesh=mesh,
        out_type=jax.ShapeDtypeStruct((B * EMBED_DIM,), jnp.float32),
        compiler_params=pltpu.CompilerParams(
            use_tc_tiling_on_sc=False, needs_layout_passes=False,
            disable_bounds_checks=True),
        scratch_types=[
            pltpu.VMEM((IHR, 128), jnp.int32),
            pltpu.VMEM((IHR, 128), jnp.int32),
            pltpu.VMEM((UNIT, EMBED_DIM), jnp.float32),
            pltpu.VMEM((UNIT, EMBED_DIM), jnp.float32),
            pltpu.VMEM((UNIT_OUT,), jnp.float32),
            pltpu.VMEM((UNIT_OUT,), jnp.float32),
            pltpu.SemaphoreType.DMA,
            pltpu.SemaphoreType.DMA,
            pltpu.SemaphoreType.DMA,
            pltpu.SemaphoreType.DMA,
            pltpu.SemaphoreType.DMA,
            pltpu.SemaphoreType.DMA,
        ],
    )
    def gather_kernel(idx_hbm, table_hbm, out_hbm, idx0, idx1, g0, g1,
                      t0, t1, sg0, sg1, sw0, sw1, si0, si1):
        wid = lax.axis_index("s") * _NUM_CORES + lax.axis_index("c")
        u0 = wid * UNITS_PER_W
        idx_b = (idx0, idx1)
        g_b = (g0, g1)
        t_b = (t0, t1)
        sg = (sg0, sg1)
        sw = (sw0, sw1)
        si = (si0, si1)
        iota = lax.iota(jnp.int32, 16)
        rot = [(iota + s) % 16 for s in range(16)]
        froti = [(r // 8) * SEG + (r % 8) * 128 + iota for r in rot]

        def unit_jir(u):
            gu = u0 + u
            return gu // UNITS_PER_COL, gu % UNITS_PER_COL

        def load_idx(u, b):
            j, ir = unit_jir(u)
            pltpu.async_copy(
                idx_hbm.at[j // 8, pl.ds(ir * IHR, IHR),
                           pl.ds((j % 8) * 128, 128)],
                idx_b[b], si[b])

        def wait_idx(b):
            pltpu.make_async_copy(
                idx_hbm.at[0, pl.ds(0, IHR), pl.ds(0, 128)], idx_b[b], si[b]
            ).wait()

        def fire_gather(b):
            for k in range(IHR):
                pltpu.async_copy(
                    table_hbm.at[idx_b[b].at[k]],
                    g_b[b].at[pl.ds(k * 128, 128)], sg[b])

        def wait_gather(b):
            for k in range(IHR):
                pltpu.make_async_copy(
                    table_hbm.at[idx_b[b].at[k]],
                    g_b[b].at[pl.ds(k * 128, 128)], sg[b]
                ).wait()

        load_idx(0, 0)
        wait_idx(0)
        fire_gather(0)
        load_idx(1, 1)

        def unit_step(u, b):
            j, ir = unit_jir(u)
            gbuf = g_b[b]
            tbuf = t_b[b]

            wait_gather(b)

            @pl.when(u + 1 < UNITS_PER_W)
            def _next_gather():
                wait_idx(1 - b)
                fire_gather(1 - b)

            @pl.when(u + 2 < UNITS_PER_W)
            def _stage_idx():
                load_idx(u + 2, b)

            @pl.when(u >= 2)
            def _drain():
                for _ in range(NDH):
                    pltpu.make_async_copy(
                        tbuf.at[pl.ds(0, SEG)], out_hbm.at[pl.ds(0, SEG)],
                        sw[b],
                    ).wait()

            @plsc.parallel_loop(0, UNIT // 16, unroll=8)
            def _transpose(mb):
                rowvec = mb * 16 + iota
                sbase = (mb // 8) * 1024 + (mb % 8) * 16
                for dg in range(2):
                    for s in range(16):
                        colvec = dg * 16 + rot[s]
                        sivec = sbase + dg * (2 * SEG) + froti[s]
                        vals = plsc.load_gather(gbuf, [rowvec, colvec])
                        plsc.store_scatter(tbuf, [sivec], vals)

            out_base = pl.multiple_of(
                j * (ROWS * EMBED_DIM) + ir * (UNIT_OUT // NDH), 8)
            for dh in range(NDH):
                pltpu.async_copy(
                    tbuf.at[pl.ds(dh * SEG, SEG)],
                    out_hbm.at[pl.ds(out_base + dh * (ROWS * EMBED_DIM
                                                      // NDH), SEG)],
                    sw[b],
                )

        def outer(g2, _):
            for b in range(2):
                unit_step(g2 * 2 + b, b)
            return 0

        lax.fori_loop(0, UNITS_PER_W // 2, outer, 0)

        for b in range(2):
            for _ in range(NDH):
                pltpu.make_async_copy(
                    t_b[b].at[pl.ds(0, SEG)], out_hbm.at[pl.ds(0, SEG)], sw[b]
                ).wait()

    return gather_kernel


_gather = _make_gather()


def kernel(sentence, table):
    idx_view = (jnp.swapaxes(sentence, 0, 1)
                .reshape(COLS // 8, 8, ROWS // 128, 128)
                .transpose(0, 2, 1, 3)
                .reshape(COLS // 8, ROWS // 128, 8 * 128)
                .astype(jnp.int32))
    out_flat = _gather(idx_view, table)
    out_q = out_flat.reshape(COLS, NDH, ROWS // 128, 8, 128)
    return out_q.transpose(2, 4, 0, 1, 3).reshape(ROWS, COLS, EMBED_DIM)


# speedup vs baseline: 1.0014x; 1.0014x over previous
import functools

import jax
import jax.numpy as jnp
from jax import lax
from jax.experimental import pallas as pl
from jax.experimental.pallas import tpu as pltpu
from jax.experimental.pallas import tpu_sc as plsc

ROWS = 4096
COLS = 200
EMBED_DIM = 32
B = ROWS * COLS

_NUM_CORES = 2
_NUM_SUBCORES = 16
NW = _NUM_CORES * _NUM_SUBCORES

UNIT = 512
IHR = UNIT // 128
NDH = EMBED_DIM // 8
UNIT_OUT = UNIT * EMBED_DIM
SEG = UNIT_OUT // NDH
UNITS_PER_COL = ROWS // UNIT
NUNITS = COLS * UNITS_PER_COL
UNITS_PER_W = NUNITS // NW


def _make_gather():
    mesh = plsc.VectorSubcoreMesh(core_axis_name="c", subcore_axis_name="s")

    @functools.partial(
        pl.kernel,
        mesh=mesh,
        out_type=jax.ShapeDtypeStruct((B * EMBED_DIM,), jnp.float32),
        compiler_params=pltpu.CompilerParams(
            use_tc_tiling_on_sc=False, needs_layout_passes=False,
            disable_bounds_checks=True),
        scratch_types=[
            pltpu.VMEM((IHR, 128), jnp.int32),
            pltpu.VMEM((IHR, 128), jnp.int32),
            pltpu.VMEM((UNIT, EMBED_DIM), jnp.float32),
            pltpu.VMEM((UNIT, EMBED_DIM), jnp.float32),
            pltpu.VMEM((UNIT_OUT,), jnp.float32),
            pltpu.VMEM((UNIT_OUT,), jnp.float32),
            pltpu.SemaphoreType.DMA,
            pltpu.SemaphoreType.DMA,
            pltpu.SemaphoreType.DMA,
            pltpu.SemaphoreType.DMA,
            pltpu.SemaphoreType.DMA,
            pltpu.SemaphoreType.DMA,
        ],
    )
    def gather_kernel(idx_hbm, table_hbm, out_hbm, idx0, idx1, g0, g1,
                      t0, t1, sg0, sg1, sw0, sw1, si0, si1):
        wid = lax.axis_index("s") * _NUM_CORES + lax.axis_index("c")
        u0 = wid * UNITS_PER_W
        idx_b = (idx0, idx1)
        g_b = (g0, g1)
        t_b = (t0, t1)
        sg = (sg0, sg1)
        sw = (sw0, sw1)
        si = (si0, si1)
        iota = lax.iota(jnp.int32, 16)
        rot = [(iota + s) % 16 for s in range(16)]
        froti = [(r // 8) * SEG + (r % 8) * 128 + iota for r in rot]

        def unit_jir(u):
            gu = u0 + u
            return gu // UNITS_PER_COL, gu % UNITS_PER_COL

        def load_idx(u, b):
            j, ir = unit_jir(u)
            pltpu.async_copy(
                idx_hbm.at[j // 8, pl.ds(ir * IHR, IHR),
                           pl.ds((j % 8) * 128, 128)],
                idx_b[b], si[b])

        def wait_idx(b):
            pltpu.make_async_copy(
                idx_hbm.at[0, pl.ds(0, IHR), pl.ds(0, 128)], idx_b[b], si[b]
            ).wait()

        def fire_gather(b):
            for k in range(IHR):
                pltpu.async_copy(
                    table_hbm.at[idx_b[b].at[k]],
                    g_b[b].at[pl.ds(k * 128, 128)], sg[b])

        def wait_gather(b):
            for k in range(IHR):
                pltpu.make_async_copy(
                    table_hbm.at[idx_b[b].at[k]],
                    g_b[b].at[pl.ds(k * 128, 128)], sg[b]
                ).wait()

        load_idx(0, 0)
        wait_idx(0)
        fire_gather(0)
        load_idx(1, 1)

        def unit_step(u, b):
            j, ir = unit_jir(u)
            gbuf = g_b[b]
            tbuf = t_b[b]

            wait_gather(b)

            @pl.when(u + 1 < UNITS_PER_W)
            def _next_gather():
                wait_idx(1 - b)
                fire_gather(1 - b)

            @pl.when(u + 2 < UNITS_PER_W)
            def _stage_idx():
                load_idx(u + 2, b)

            @pl.when(u >= 2)
            def _drain():
                for _ in range(NDH):
                    pltpu.make_async_copy(
                        tbuf.at[pl.ds(0, SEG)], out_hbm.at[pl.ds(0, SEG)],
                        sw[b],
                    ).wait()

            @plsc.parallel_loop(0, UNIT // 16, unroll=4)
            def _transpose(mb):
                rowvec = mb * 16 + iota
                sbase = (mb // 8) * 1024 + (mb % 8) * 16
                for dg in range(2):
                    for s in range(16):
                        colvec = dg * 16 + rot[s]
                        sivec = sbase + dg * (2 * SEG) + froti[s]
                        vals = plsc.load_gather(gbuf, [rowvec, colvec])
                        plsc.store_scatter(tbuf, [sivec], vals)

            out_base = pl.multiple_of(
                j * (ROWS * EMBED_DIM) + ir * (UNIT_OUT // NDH), 8)
            for dh in range(NDH):
                pltpu.async_copy(
                    tbuf.at[pl.ds(dh * SEG, SEG)],
                    out_hbm.at[pl.ds(out_base + dh * (ROWS * EMBED_DIM
                                                      // NDH), SEG)],
                    sw[b],
                )

        def outer(g2, _):
            for b in range(2):
                unit_step(g2 * 2 + b, b)
            return 0

        lax.fori_loop(0, UNITS_PER_W // 2, outer, 0)

        for b in range(2):
            for _ in range(NDH):
                pltpu.make_async_copy(
                    t_b[b].at[pl.ds(0, SEG)], out_hbm.at[pl.ds(0, SEG)], sw[b]
                ).wait()

    return gather_kernel


_gather = _make_gather()


def kernel(sentence, table):
    idx_view = (jnp.swapaxes(sentence, 0, 1)
                .reshape(COLS // 8, 8, ROWS // 128, 128)
                .transpose(0, 2, 1, 3)
                .reshape(COLS // 8, ROWS // 128, 8 * 128)
                .astype(jnp.int32))
    out_flat = _gather(idx_view, table)
    out_q = out_flat.reshape(COLS, NDH, ROWS // 128, 8, 128)
    return out_q.transpose(2, 4, 0, 1, 3).reshape(ROWS, COLS, EMBED_DIM)
